# baseline (speedup 1.0000x reference)
import functools

import jax
import jax.numpy as jnp
from jax import lax
from jax.experimental import pallas as pl
from jax.experimental.pallas import tpu as pltpu
from jax.experimental.pallas import tpu_sc as plsc

_NC = 2
_NS = 16
_NW = _NC * _NS

_B, _L, _D = 16384, 200, 64
_ROWS = _B * _L
_LANES = 16
_NG = _D // _LANES
_ICHUNK = 1024
_WCH = 256
_NWCH = _ICHUNK // _WCH
_ROWS_PER_W = _ROWS // _NW
_NOUTER = _ROWS_PER_W // _ICHUNK


def _sc_body(idx_hbm, table_hbm, out_hbm, idx_v, tab_v, rows_a, rows_b, sem_a,
             sem_b):
    w = lax.axis_index("s") * _NC + lax.axis_index("c")
    row_base = w * _ROWS_PER_W

    pltpu.sync_copy(table_hbm, tab_v)
    t0 = [tab_v[0, pl.ds(g * _LANES, _LANES)] for g in range(_NG)]
    t1 = [tab_v[1, pl.ds(g * _LANES, _LANES)] for g in range(_NG)]
    bufs = [(rows_a, sem_a), (rows_b, sem_b)]

    def outer(o, carry):
        row0 = pl.multiple_of(row_base + o * _ICHUNK, _ICHUNK)
        pltpu.sync_copy(
            idx_hbm.at[pl.ds(pl.multiple_of(row0 // 128, _ICHUNK // 128),
                             _ICHUNK // 128)],
            idx_v,
        )

        for b in range(_NWCH):
            rows_v, sem = bufs[b % 2]
            dst = out_hbm.at[pl.ds(row0 + b * _WCH, _WCH)]

            if b >= 2:
                pltpu.make_async_copy(rows_v, dst, sem).wait()
            else:
                @pl.when(o > 0)
                def _():
                    pltpu.make_async_copy(rows_v, dst, sem).wait()

            def fill(r, carry_in):
                rr = b * _WCH + r
                rsplat = lax.broadcast(rr // 128, (_LANES,))
                csplat = lax.broadcast(rr % 128, (_LANES,))
                iv = plsc.load_gather(idx_v, [rsplat, csplat])
                m = iv != 0
                for g in range(_NG):
                    rows_v[r, pl.ds(g * _LANES, _LANES)] = jnp.where(
                        m, t1[g], t0[g])
                return carry_in

            lax.fori_loop(0, _WCH, fill, 0, unroll=8)
            pltpu.async_copy(rows_v, dst, sem)
        return carry

    lax.fori_loop(0, _NOUTER, outer, 0)
    for rows_v, sem in bufs:
        pltpu.make_async_copy(
            rows_v, out_hbm.at[pl.ds(0, _WCH)], sem).wait()


@jax.jit
def _sc_lookup(idx2d, table):
    mesh = plsc.VectorSubcoreMesh(
        core_axis_name="c", subcore_axis_name="s", num_cores=_NC,
        num_subcores=_NS,
    )
    fn = pl.kernel(
        _sc_body,
        out_type=jax.ShapeDtypeStruct((_ROWS, _D), jnp.float32),
        mesh=mesh,
        scratch_types=[
            pltpu.VMEM((_ICHUNK // 128, 128), jnp.int32),
            pltpu.VMEM((2, _D), jnp.float32),
            pltpu.VMEM((_WCH, _D), jnp.float32),
            pltpu.VMEM((_WCH, _D), jnp.float32),
            pltpu.SemaphoreType.DMA,
            pltpu.SemaphoreType.DMA,
        ],
        compiler_params=pltpu.CompilerParams(needs_layout_passes=False),
        name="sc_embed_lookup",
    )
    return fn(idx2d, table)


def kernel(lang, embedding_table):
    idx2d = lang.astype(jnp.int32).reshape(_ROWS // 128, 128)
    out = _sc_lookup(idx2d, embedding_table)
    return out.reshape(_B, _L, _D)


# speedup vs baseline: 1.0769x; 1.0769x over previous
import functools

import jax
import jax.numpy as jnp
from jax import lax
from jax.experimental import pallas as pl
from jax.experimental.pallas import tpu as pltpu
from jax.experimental.pallas import tpu_sc as plsc

_NC = 2
_NS = 16
_NW = _NC * _NS

_B, _L, _D = 16384, 200, 64
_ROWS = _B * _L
_LANES = 16
_NG = _D // _LANES
_ICHUNK = 1024
_WCH = 256
_NWCH = _ICHUNK // _WCH
_ROWS_PER_W = _ROWS // _NW
_NOUTER = _ROWS_PER_W // _ICHUNK


def _sc_body(idx_hbm, table_hbm, out_hbm, idx_a, idx_b, tab_v, rows_a, rows_b,
             sem_a, sem_b, isem_a, isem_b):
    w = lax.axis_index("s") * _NC + lax.axis_index("c")
    row_base = w * _ROWS_PER_W
    _IR = _ICHUNK // 128

    pltpu.sync_copy(table_hbm, tab_v)
    t0 = [tab_v[0, pl.ds(g * _LANES, _LANES)] for g in range(_NG)]
    t1 = [tab_v[1, pl.ds(g * _LANES, _LANES)] for g in range(_NG)]
    bufs = [(rows_a, sem_a), (rows_b, sem_b)]
    ibufs = [(idx_a, isem_a), (idx_b, isem_b)]

    def _idx_src(o):
        r0 = pl.multiple_of((row_base + (o % _NOUTER) * _ICHUNK) // 128, _IR)
        return idx_hbm.at[pl.ds(r0, _IR)]

    pltpu.async_copy(_idx_src(0), idx_a, isem_a)

    def outer(m, carry):
        for h in range(2):
            o = 2 * m + h
            row0 = pl.multiple_of(row_base + o * _ICHUNK, _ICHUNK)
            idx_v, isem = ibufs[h]
            idx_n, isem_n = ibufs[1 - h]
            pltpu.make_async_copy(_idx_src(o), idx_v, isem).wait()
            pltpu.async_copy(_idx_src(o + 1), idx_n, isem_n)

            for b in range(_NWCH):
                rows_v, sem = bufs[b % 2]
                dst = out_hbm.at[pl.ds(row0 + b * _WCH, _WCH)]

                if h > 0 or b >= 2:
                    pltpu.make_async_copy(rows_v, dst, sem).wait()
                else:
                    @pl.when(o > 0)
                    def _():
                        pltpu.make_async_copy(rows_v, dst, sem).wait()

                def fill(r, carry_in):
                    rr = b * _WCH + r
                    rsplat = lax.broadcast(rr // 128, (_LANES,))
                    csplat = lax.broadcast(rr % 128, (_LANES,))
                    iv = plsc.load_gather(idx_v, [rsplat, csplat])
                    mask = iv != 0
                    for g in range(_NG):
                        rows_v[r, pl.ds(g * _LANES, _LANES)] = jnp.where(
                            mask, t1[g], t0[g])
                    return carry_in

                lax.fori_loop(0, _WCH, fill, 0, unroll=4)
                pltpu.async_copy(rows_v, dst, sem)
        return carry

    lax.fori_loop(0, _NOUTER // 2, outer, 0)
    for rows_v, sem in bufs:
        pltpu.make_async_copy(
            rows_v, out_hbm.at[pl.ds(0, _WCH)], sem).wait()
    pltpu.make_async_copy(_idx_src(0), idx_a, isem_a).wait()


@jax.jit
def _sc_lookup(idx2d, table):
    mesh = plsc.VectorSubcoreMesh(
        core_axis_name="c", subcore_axis_name="s", num_cores=_NC,
        num_subcores=_NS,
    )
    fn = pl.kernel(
        _sc_body,
        out_type=jax.ShapeDtypeStruct((_ROWS, _D), jnp.float32),
        mesh=mesh,
        scratch_types=[
            pltpu.VMEM((_ICHUNK // 128, 128), jnp.int32),
            pltpu.VMEM((_ICHUNK // 128, 128), jnp.int32),
            pltpu.VMEM((2, _D), jnp.float32),
            pltpu.VMEM((_WCH, _D), jnp.float32),
            pltpu.VMEM((_WCH, _D), jnp.float32),
            pltpu.SemaphoreType.DMA,
            pltpu.SemaphoreType.DMA,
            pltpu.SemaphoreType.DMA,
            pltpu.SemaphoreType.DMA,
        ],
        compiler_params=pltpu.CompilerParams(needs_layout_passes=False),
        name="sc_embed_lookup",
    )
    return fn(idx2d, table)


def kernel(lang, embedding_table):
    idx2d = lang.astype(jnp.int32).reshape(_ROWS // 128, 128)
    out = _sc_lookup(idx2d, embedding_table)
    return out.reshape(_B, _L, _D)
